# baseline (speedup 1.0000x reference)
import functools

import jax
import jax.numpy as jnp
from jax import lax
from jax.experimental import pallas as pl
from jax.experimental.pallas import tpu as pltpu
from jax.experimental.pallas import tpu_sc as plsc

N = 10000
E = 320000
D = 128
NPAD = 10240

NC = 2
NS = 16
NW = NC * NS
EW = E // NW
CH = 128
SLABC = 16
NSLAB = 5
NCHUNK = NSLAB * SLABC
EWP = NCHUNK * CH
RPS = NPAD // NS
ZCH = 128
NZ = RPS // ZCH

_MESH = dict(core_axis_name="c", subcore_axis_name="s",
             num_cores=NC, num_subcores=NS)


def _msg_body(h, src4, dst4, zrow_h, P,
              acc, src_v, dst_v, rows_v, sem):
    cid = lax.axis_index("c")
    sid = lax.axis_index("s")
    wid = cid * NS + sid

    pltpu.sync_copy(src4.at[wid], src_v)
    pltpu.sync_copy(dst4.at[wid], dst_v)

    pltpu.sync_copy(zrow_h, rows_v)
    for i in range(NZ):
        pltpu.sync_copy(rows_v, acc.at[pl.ds(sid * RPS + i * ZCH, ZCH)])

    plsc.subcore_barrier()

    def chunk(j, _):
        pltpu.async_copy(h.at[src_v.at[j]], rows_v, sem).wait()
        pltpu.sync_copy(rows_v, acc.at[dst_v.at[j]], add=True)
        return ()

    lax.fori_loop(0, NCHUNK, chunk, (), unroll=False)

    plsc.subcore_barrier()

    for i in range(NZ):
        r0 = sid * RPS + i * ZCH
        pltpu.sync_copy(acc.at[pl.ds(r0, ZCH)], rows_v)
        pltpu.sync_copy(rows_v, P.at[cid, pl.ds(r0, ZCH)])


@functools.lru_cache(maxsize=None)
def _get_sc_msg():
  return pl.kernel(
    _msg_body,
    out_type=jax.ShapeDtypeStruct((NC, NPAD, D), jnp.float32),
    mesh=plsc.VectorSubcoreMesh(**_MESH),
    scratch_types=[
        pltpu.VMEM_SHARED((NPAD, D), jnp.float32),
        pltpu.VMEM((NCHUNK, CH), jnp.int32),
        pltpu.VMEM((NCHUNK, CH), jnp.int32),
        pltpu.VMEM((ZCH, D), jnp.float32),
        pltpu.SemaphoreType.DMA,
    ],
  )


def _deg_body(dst3, zrow_h, DEG, dst_v, hist_v):
    cid = lax.axis_index("c")
    sid = lax.axis_index("s")
    wid = cid * NS + sid

    pltpu.sync_copy(dst3.at[wid], dst_v)
    pltpu.sync_copy(zrow_h.at[pl.ds(0, NPAD // D)], hist_v)

    ones = jnp.ones((16,), jnp.float32)

    def chunk(j, _):
        for k in range(CH // 16):
            dv = dst_v[j, pl.ds(k * 16, 16)]
            hi = lax.shift_right_logical(dv, 7)
            lo = lax.bitwise_and(dv, 127)
            plsc.addupdate_scatter(hist_v, [hi, lo], ones)
        return ()

    lax.fori_loop(0, NCHUNK, chunk, (), unroll=False)

    pltpu.sync_copy(hist_v, DEG.at[wid])


@functools.lru_cache(maxsize=None)
def _get_sc_deg():
  return pl.kernel(
    _deg_body,
    out_type=jax.ShapeDtypeStruct((NW, NPAD // D, D), jnp.float32),
    mesh=plsc.VectorSubcoreMesh(**_MESH),
    scratch_types=[
        pltpu.VMEM((NCHUNK, CH), jnp.int32),
        pltpu.VMEM((NPAD // D, D), jnp.float32),
    ],
    compiler_params=pltpu.CompilerParams(needs_layout_passes=False),
  )


def _tc_body(relu, x_ref, p_ref, deg_ref, ws_ref, wn_ref, b_ref, o_ref):
    s = p_ref[0] + p_ref[1]
    d = jnp.sum(deg_ref[...], axis=0)
    hn = s / jnp.maximum(d, 1.0)[:, None]
    o = (jnp.dot(x_ref[...], ws_ref[...], preferred_element_type=jnp.float32)
         + jnp.dot(hn, wn_ref[...], preferred_element_type=jnp.float32)
         + b_ref[...])
    if relu:
        o = jnp.maximum(o, 0.0)
    o_ref[...] = o


_TCB = 512


def _tc_dense(x, P, DEG, W_self, W_neigh, b, relu):
    grid = (NPAD // _TCB,)
    return pl.pallas_call(
        functools.partial(_tc_body, relu),
        grid=grid,
        in_specs=[
            pl.BlockSpec((_TCB, D), lambda i: (i, 0)),
            pl.BlockSpec((NC, _TCB, D), lambda i: (0, i, 0)),
            pl.BlockSpec((NW, _TCB), lambda i: (0, i)),
            pl.BlockSpec((D, D), lambda i: (0, 0)),
            pl.BlockSpec((D, D), lambda i: (0, 0)),
            pl.BlockSpec((1, D), lambda i: (0, 0)),
        ],
        out_specs=pl.BlockSpec((_TCB, D), lambda i: (i, 0)),
        out_shape=jax.ShapeDtypeStruct((NPAD, D), jnp.float32),
    )(x, P, DEG, W_self, W_neigh, b.reshape(1, D))


def kernel(inputs, edge_index, W_self0, W_neigh0, b0, W_self1, W_neigh1, b1):
    pad = EWP - EW
    srcw = jnp.concatenate(
        [edge_index[0].reshape(NW, EW),
         jnp.zeros((NW, pad), jnp.int32)], axis=1)
    dpad = N + (jnp.arange(pad, dtype=jnp.int32) % (NPAD - N))
    dstw = jnp.concatenate(
        [edge_index[1].reshape(NW, EW),
         jnp.broadcast_to(dpad, (NW, pad))], axis=1)
    src4 = srcw.reshape(NW, NCHUNK, CH)
    dst4 = dstw.reshape(NW, NCHUNK, CH)
    dst3 = dstw.reshape(NW, NCHUNK, CH)
    zrow = jnp.zeros((ZCH, D), jnp.float32)

    xp = jnp.zeros((NPAD, D), jnp.float32).at[:N].set(inputs)
    DEG = _get_sc_deg()(dst3, zrow).reshape(NW, NPAD)
    P0 = _get_sc_msg()(xp, src4, dst4, zrow)
    h1 = _tc_dense(xp, P0, DEG, W_self0, W_neigh0, b0, relu=True)
    P1 = _get_sc_msg()(h1, src4, dst4, zrow)
    out = _tc_dense(h1, P1, DEG, W_self1, W_neigh1, b1, relu=False)
    return out[:N]


# speedup vs baseline: 1.9985x; 1.9985x over previous
import functools

import jax
import jax.numpy as jnp
from jax import lax
from jax.experimental import pallas as pl
from jax.experimental.pallas import tpu as pltpu
from jax.experimental.pallas import tpu_sc as plsc

N = 10000
E = 320000
D = 128
NPAD = 10240

NC = 2
NS = 16
NW = NC * NS
EW = E // NW
CH = 80
NCHUNK = EW // CH
RPS = NPAD // NS
ZCH = CH
NZ = RPS // ZCH

_MESH = dict(core_axis_name="c", subcore_axis_name="s",
             num_cores=NC, num_subcores=NS)


def _msg_body(h, src4, dst4, zrow_h, P,
              acc, src_v, dst_v, rows_v, sem):
    cid = lax.axis_index("c")
    sid = lax.axis_index("s")
    wid = cid * NS + sid

    pltpu.sync_copy(src4.at[wid], src_v)
    pltpu.sync_copy(dst4.at[wid], dst_v)

    pltpu.sync_copy(zrow_h, rows_v)
    for i in range(NZ):
        pltpu.sync_copy(rows_v, acc.at[pl.ds(sid * RPS + i * ZCH, ZCH)])

    plsc.subcore_barrier()

    def chunk(j, _):
        pltpu.async_copy(h.at[src_v.at[j]], rows_v, sem).wait()
        pltpu.sync_copy(rows_v, acc.at[dst_v.at[j]], add=True)
        return ()

    lax.fori_loop(0, NCHUNK, chunk, (), unroll=False)

    plsc.subcore_barrier()

    for i in range(NZ):
        r0 = sid * RPS + i * ZCH
        pltpu.sync_copy(acc.at[pl.ds(r0, ZCH)], rows_v)
        pltpu.sync_copy(rows_v, P.at[cid, pl.ds(r0, ZCH)])


@functools.lru_cache(maxsize=None)
def _get_sc_msg():
  return pl.kernel(
    _msg_body,
    out_type=jax.ShapeDtypeStruct((NC, NPAD, D), jnp.float32),
    mesh=plsc.VectorSubcoreMesh(**_MESH),
    scratch_types=[
        pltpu.VMEM_SHARED((NPAD, D), jnp.float32),
        pltpu.VMEM((NCHUNK, CH), jnp.int32),
        pltpu.VMEM((NCHUNK, CH), jnp.int32),
        pltpu.VMEM((ZCH, D), jnp.float32),
        pltpu.SemaphoreType.DMA,
    ],
  )


def _deg_body(dst3, zrow_h, DEG, dst_v, hist_v):
    cid = lax.axis_index("c")
    sid = lax.axis_index("s")
    wid = cid * NS + sid

    pltpu.sync_copy(dst3.at[wid], dst_v)
    pltpu.sync_copy(zrow_h.at[pl.ds(0, NPAD // D)], hist_v)

    ones = jnp.ones((16,), jnp.float32)

    def chunk(j, _):
        for k in range(CH // 16):
            dv = dst_v[j, pl.ds(k * 16, 16)]
            hi = lax.shift_right_logical(dv, 7)
            lo = lax.bitwise_and(dv, 127)
            plsc.addupdate_scatter(hist_v, [hi, lo], ones)
        return ()

    lax.fori_loop(0, NCHUNK, chunk, (), unroll=False)

    pltpu.sync_copy(hist_v, DEG.at[wid])


@functools.lru_cache(maxsize=None)
def _get_sc_deg():
  return pl.kernel(
    _deg_body,
    out_type=jax.ShapeDtypeStruct((NW, NPAD // D, D), jnp.float32),
    mesh=plsc.VectorSubcoreMesh(**_MESH),
    scratch_types=[
        pltpu.VMEM((NCHUNK, CH), jnp.int32),
        pltpu.VMEM((NPAD // D, D), jnp.float32),
    ],
    compiler_params=pltpu.CompilerParams(needs_layout_passes=False),
  )


def _tc_body(relu, x_ref, p_ref, deg_ref, ws_ref, wn_ref, b_ref, o_ref):
    s = p_ref[0] + p_ref[1]
    d = jnp.sum(deg_ref[...], axis=0)
    hn = s / jnp.maximum(d, 1.0)[:, None]
    o = (jnp.dot(x_ref[...], ws_ref[...], preferred_element_type=jnp.float32)
         + jnp.dot(hn, wn_ref[...], preferred_element_type=jnp.float32)
         + b_ref[...])
    if relu:
        o = jnp.maximum(o, 0.0)
    o_ref[...] = o


_TCB = 512


def _tc_dense(x, P, DEG, W_self, W_neigh, b, relu):
    grid = (NPAD // _TCB,)
    return pl.pallas_call(
        functools.partial(_tc_body, relu),
        grid=grid,
        in_specs=[
            pl.BlockSpec((_TCB, D), lambda i: (i, 0)),
            pl.BlockSpec((NC, _TCB, D), lambda i: (0, i, 0)),
            pl.BlockSpec((NW, _TCB), lambda i: (0, i)),
            pl.BlockSpec((D, D), lambda i: (0, 0)),
            pl.BlockSpec((D, D), lambda i: (0, 0)),
            pl.BlockSpec((1, D), lambda i: (0, 0)),
        ],
        out_specs=pl.BlockSpec((_TCB, D), lambda i: (i, 0)),
        out_shape=jax.ShapeDtypeStruct((NPAD, D), jnp.float32),
    )(x, P, DEG, W_self, W_neigh, b.reshape(1, D))


def kernel(inputs, edge_index, W_self0, W_neigh0, b0, W_self1, W_neigh1, b1):
    src4 = edge_index[0].reshape(NW, NCHUNK, CH)
    dst4 = edge_index[1].reshape(NW, NCHUNK, CH)
    dst3 = dst4
    zrow = jnp.zeros((ZCH, D), jnp.float32)

    xp = jnp.zeros((NPAD, D), jnp.float32).at[:N].set(inputs)
    DEG = _get_sc_deg()(dst3, zrow).reshape(NW, NPAD)
    P0 = _get_sc_msg()(xp, src4, dst4, zrow)
    h1 = _tc_dense(xp, P0, DEG, W_self0, W_neigh0, b0, relu=True)
    P1 = _get_sc_msg()(h1, src4, dst4, zrow)
    out = _tc_dense(h1, P1, DEG, W_self1, W_neigh1, b1, relu=False)
    return out[:N]


# speedup vs baseline: 2.9017x; 1.4519x over previous
import functools

import jax
import jax.numpy as jnp
from jax import lax
from jax.experimental import pallas as pl
from jax.experimental.pallas import tpu as pltpu
from jax.experimental.pallas import tpu_sc as plsc

N = 10000
E = 320000
D = 128
NPAD = 10240

NC = 2
NS = 16
NW = NC * NS
EW = E // NW
CH = 80
NCHUNK = EW // CH
SLABC = 25
NSLAB = NCHUNK // SLABC
RPS = NPAD // NS
ZCH = CH
NZ = RPS // ZCH

_MESH = dict(core_axis_name="c", subcore_axis_name="s",
             num_cores=NC, num_subcores=NS)


def _msg_body(h, src4, dst4, zrow_h, P,
              acc, src_a, src_b, dst_a, dst_b, rows_a, rows_b,
              sem_a, sem_b):
    cid = lax.axis_index("c")
    sid = lax.axis_index("s")
    wid = cid * NS + sid

    pltpu.sync_copy(zrow_h, rows_a)
    for i in range(NZ):
        pltpu.sync_copy(rows_a, acc.at[pl.ds(sid * RPS + i * ZCH, ZCH)])

    plsc.subcore_barrier()

    for s in range(NSLAB):
        src_s = src_a if s % 2 == 0 else src_b
        dst_s = dst_a if s % 2 == 0 else dst_b
        pltpu.sync_copy(src4.at[wid, s], src_s)
        pltpu.sync_copy(dst4.at[wid, s], dst_s)

        pltpu.async_copy(h.at[src_s.at[0]], rows_a, sem_a)

        def pair(i, _, src_s=src_s, dst_s=dst_s):
            c1 = 2 * i + 1
            pltpu.async_copy(h.at[src_s.at[c1]], rows_b, sem_b)
            pltpu.make_async_copy(h.at[src_s.at[2 * i]], rows_a, sem_a).wait()
            pltpu.sync_copy(rows_a, acc.at[dst_s.at[2 * i]], add=True)
            pltpu.async_copy(h.at[src_s.at[c1 + 1]], rows_a, sem_a)
            pltpu.make_async_copy(h.at[src_s.at[c1]], rows_b, sem_b).wait()
            pltpu.sync_copy(rows_b, acc.at[dst_s.at[c1]], add=True)
            return ()

        lax.fori_loop(0, SLABC // 2, pair, (), unroll=False)

        pltpu.make_async_copy(
            h.at[src_s.at[SLABC - 1]], rows_a, sem_a).wait()
        pltpu.sync_copy(rows_a, acc.at[dst_s.at[SLABC - 1]], add=True)

    plsc.subcore_barrier()

    for i in range(NZ):
        r0 = sid * RPS + i * ZCH
        pltpu.sync_copy(acc.at[pl.ds(r0, ZCH)], rows_a)
        pltpu.sync_copy(rows_a, P.at[cid, pl.ds(r0, ZCH)])


@functools.lru_cache(maxsize=None)
def _get_sc_msg():
  return pl.kernel(
    _msg_body,
    out_type=jax.ShapeDtypeStruct((NC, NPAD, D), jnp.float32),
    mesh=plsc.VectorSubcoreMesh(**_MESH),
    scratch_types=[
        pltpu.VMEM_SHARED((NPAD, D), jnp.float32),
        pltpu.VMEM((SLABC, CH), jnp.int32),
        pltpu.VMEM((SLABC, CH), jnp.int32),
        pltpu.VMEM((SLABC, CH), jnp.int32),
        pltpu.VMEM((SLABC, CH), jnp.int32),
        pltpu.VMEM((ZCH, D), jnp.float32),
        pltpu.VMEM((ZCH, D), jnp.float32),
        pltpu.SemaphoreType.DMA,
        pltpu.SemaphoreType.DMA,
    ],
  )


def _deg_body(dst3, zrow_h, DEG, dst_v, hist_v):
    cid = lax.axis_index("c")
    sid = lax.axis_index("s")
    wid = cid * NS + sid

    pltpu.sync_copy(dst3.at[wid], dst_v)
    pltpu.sync_copy(zrow_h.at[pl.ds(0, NPAD // D)], hist_v)

    ones = jnp.ones((16,), jnp.float32)

    def chunk(j, _):
        for k in range(CH // 16):
            dv = dst_v[j, pl.ds(k * 16, 16)]
            hi = lax.shift_right_logical(dv, 7)
            lo = lax.bitwise_and(dv, 127)
            plsc.addupdate_scatter(hist_v, [hi, lo], ones)
        return ()

    lax.fori_loop(0, NCHUNK, chunk, (), unroll=False)

    pltpu.sync_copy(hist_v, DEG.at[wid])


@functools.lru_cache(maxsize=None)
def _get_sc_deg():
  return pl.kernel(
    _deg_body,
    out_type=jax.ShapeDtypeStruct((NW, NPAD // D, D), jnp.float32),
    mesh=plsc.VectorSubcoreMesh(**_MESH),
    scratch_types=[
        pltpu.VMEM((NCHUNK, CH), jnp.int32),
        pltpu.VMEM((NPAD // D, D), jnp.float32),
    ],
    compiler_params=pltpu.CompilerParams(needs_layout_passes=False),
  )


def _tc_body(relu, x_ref, p_ref, deg_ref, ws_ref, wn_ref, b_ref, o_ref):
    s = p_ref[0] + p_ref[1]
    d = jnp.sum(deg_ref[...], axis=0)
    hn = s / jnp.maximum(d, 1.0)[:, None]
    o = (jnp.dot(x_ref[...], ws_ref[...], preferred_element_type=jnp.float32)
         + jnp.dot(hn, wn_ref[...], preferred_element_type=jnp.float32)
         + b_ref[...])
    if relu:
        o = jnp.maximum(o, 0.0)
    o_ref[...] = o


_TCB = 512


def _tc_dense(x, P, DEG, W_self, W_neigh, b, relu):
    grid = (NPAD // _TCB,)
    return pl.pallas_call(
        functools.partial(_tc_body, relu),
        grid=grid,
        in_specs=[
            pl.BlockSpec((_TCB, D), lambda i: (i, 0)),
            pl.BlockSpec((NC, _TCB, D), lambda i: (0, i, 0)),
            pl.BlockSpec((NW, _TCB), lambda i: (0, i)),
            pl.BlockSpec((D, D), lambda i: (0, 0)),
            pl.BlockSpec((D, D), lambda i: (0, 0)),
            pl.BlockSpec((1, D), lambda i: (0, 0)),
        ],
        out_specs=pl.BlockSpec((_TCB, D), lambda i: (i, 0)),
        out_shape=jax.ShapeDtypeStruct((NPAD, D), jnp.float32),
    )(x, P, DEG, W_self, W_neigh, b.reshape(1, D))


def kernel(inputs, edge_index, W_self0, W_neigh0, b0, W_self1, W_neigh1, b1):
    src4 = edge_index[0].reshape(NW, NSLAB, SLABC, CH)
    dst4 = edge_index[1].reshape(NW, NSLAB, SLABC, CH)
    dst3 = edge_index[1].reshape(NW, NCHUNK, CH)
    zrow = jnp.zeros((ZCH, D), jnp.float32)

    xp = jnp.zeros((NPAD, D), jnp.float32).at[:N].set(inputs)
    DEG = _get_sc_deg()(dst3, zrow).reshape(NW, NPAD)
    P0 = _get_sc_msg()(xp, src4, dst4, zrow)
    h1 = _tc_dense(xp, P0, DEG, W_self0, W_neigh0, b0, relu=True)
    P1 = _get_sc_msg()(h1, src4, dst4, zrow)
    out = _tc_dense(h1, P1, DEG, W_self1, W_neigh1, b1, relu=False)
    return out[:N]


# speedup vs baseline: 3.0327x; 1.0452x over previous
import functools

import jax
import jax.numpy as jnp
from jax import lax
from jax.experimental import pallas as pl
from jax.experimental.pallas import tpu as pltpu
from jax.experimental.pallas import tpu_sc as plsc

N = 10000
E = 320000
D = 128
NPAD = 10240

NC = 2
NS = 16
NW = NC * NS
EW = E // NW
CH = 80
NCHUNK = EW // CH
SLABC = 25
NSLAB = NCHUNK // SLABC
RPS = NPAD // NS
ZCH = CH
NZ = RPS // ZCH

_MESH = dict(core_axis_name="c", subcore_axis_name="s",
             num_cores=NC, num_subcores=NS)


def _msg_body(h, src4, dst4, zrow_h, P,
              acc, src_a, src_b, dst_a, dst_b, rows_a, rows_b,
              sem_a, sem_b):
    cid = lax.axis_index("c")
    sid = lax.axis_index("s")
    wid = cid * NS + sid

    pltpu.sync_copy(zrow_h, rows_a)
    for i in range(NZ):
        pltpu.sync_copy(rows_a, acc.at[pl.ds(sid * RPS + i * ZCH, ZCH)])

    plsc.subcore_barrier()

    for s in range(NSLAB):
        src_s = src_a if s % 2 == 0 else src_b
        dst_s = dst_a if s % 2 == 0 else dst_b
        pltpu.sync_copy(src4.at[wid, s], src_s)
        pltpu.sync_copy(dst4.at[wid, s], dst_s)

        pltpu.async_copy(h.at[src_s.at[0]], rows_a, sem_a)

        def pair(i, _, src_s=src_s, dst_s=dst_s):
            c1 = 2 * i + 1
            pltpu.async_copy(h.at[src_s.at[c1]], rows_b, sem_b)
            pltpu.make_async_copy(h.at[src_s.at[2 * i]], rows_a, sem_a).wait()
            pltpu.sync_copy(rows_a, acc.at[dst_s.at[2 * i]], add=True)
            pltpu.async_copy(h.at[src_s.at[c1 + 1]], rows_a, sem_a)
            pltpu.make_async_copy(h.at[src_s.at[c1]], rows_b, sem_b).wait()
            pltpu.sync_copy(rows_b, acc.at[dst_s.at[c1]], add=True)
            return ()

        lax.fori_loop(0, SLABC // 2, pair, (), unroll=False)

        pltpu.make_async_copy(
            h.at[src_s.at[SLABC - 1]], rows_a, sem_a).wait()
        pltpu.sync_copy(rows_a, acc.at[dst_s.at[SLABC - 1]], add=True)

    plsc.subcore_barrier()

    for i in range(NZ):
        r0 = sid * RPS + i * ZCH
        pltpu.sync_copy(acc.at[pl.ds(r0, ZCH)], rows_a)
        pltpu.sync_copy(rows_a, P.at[cid, pl.ds(r0, ZCH)])


@functools.lru_cache(maxsize=None)
def _get_sc_msg():
  return pl.kernel(
    _msg_body,
    out_type=jax.ShapeDtypeStruct((NC, NPAD, D), jnp.float32),
    mesh=plsc.VectorSubcoreMesh(**_MESH),
    scratch_types=[
        pltpu.VMEM_SHARED((NPAD, D), jnp.float32),
        pltpu.VMEM((SLABC, CH), jnp.int32),
        pltpu.VMEM((SLABC, CH), jnp.int32),
        pltpu.VMEM((SLABC, CH), jnp.int32),
        pltpu.VMEM((SLABC, CH), jnp.int32),
        pltpu.VMEM((ZCH, D), jnp.float32),
        pltpu.VMEM((ZCH, D), jnp.float32),
        pltpu.SemaphoreType.DMA,
        pltpu.SemaphoreType.DMA,
    ],
  )


def _deg_body(dst3, zrow_h, DEG, dst_v, hist_v):
    cid = lax.axis_index("c")
    sid = lax.axis_index("s")
    wid = cid * NS + sid

    pltpu.sync_copy(dst3.at[wid], dst_v)
    pltpu.sync_copy(zrow_h.at[pl.ds(0, NPAD // D)], hist_v)

    ones = jnp.ones((16,), jnp.float32)

    def chunk(j, _):
        for k in range(CH // 16):
            dv = dst_v[j, pl.ds(k * 16, 16)]
            hi = lax.shift_right_logical(dv, 7)
            lo = lax.bitwise_and(dv, 127)
            plsc.addupdate_scatter(hist_v, [hi, lo], ones)
        return ()

    lax.fori_loop(0, NCHUNK, chunk, (), unroll=False)

    pltpu.sync_copy(hist_v, DEG.at[wid])


@functools.lru_cache(maxsize=None)
def _get_sc_deg():
  return pl.kernel(
    _deg_body,
    out_type=jax.ShapeDtypeStruct((NW, NPAD // D, D), jnp.float32),
    mesh=plsc.VectorSubcoreMesh(**_MESH),
    scratch_types=[
        pltpu.VMEM((NCHUNK, CH), jnp.int32),
        pltpu.VMEM((NPAD // D, D), jnp.float32),
    ],
    compiler_params=pltpu.CompilerParams(needs_layout_passes=False),
  )


def _tc_body(relu, x_ref, p_ref, deg_ref, ws_ref, wn_ref, b_ref, o_ref):
    s = p_ref[0] + p_ref[1]
    d = jnp.sum(deg_ref[...], axis=1)
    hn = s / jnp.maximum(d, 1.0)[:, None]
    o = (jnp.dot(x_ref[...], ws_ref[...], preferred_element_type=jnp.float32)
         + jnp.dot(hn, wn_ref[...], preferred_element_type=jnp.float32)
         + b_ref[...])
    if relu:
        o = jnp.maximum(o, 0.0)
    o_ref[...] = o


_TCB = 1000


def _tc_dense(x, P, DEG, W_self, W_neigh, b, relu):
    grid = (N // _TCB,)
    return pl.pallas_call(
        functools.partial(_tc_body, relu),
        grid=grid,
        in_specs=[
            pl.BlockSpec((_TCB, D), lambda i: (i, 0)),
            pl.BlockSpec((NC, _TCB, D), lambda i: (0, i, 0)),
            pl.BlockSpec((_TCB, NW), lambda i: (i, 0)),
            pl.BlockSpec((D, D), lambda i: (0, 0)),
            pl.BlockSpec((D, D), lambda i: (0, 0)),
            pl.BlockSpec((1, D), lambda i: (0, 0)),
        ],
        out_specs=pl.BlockSpec((_TCB, D), lambda i: (i, 0)),
        out_shape=jax.ShapeDtypeStruct((N, D), jnp.float32),
    )(x, P, DEG, W_self, W_neigh, b.reshape(1, D))


def kernel(inputs, edge_index, W_self0, W_neigh0, b0, W_self1, W_neigh1, b1):
    src4 = edge_index[0].reshape(NW, NSLAB, SLABC, CH)
    dst4 = edge_index[1].reshape(NW, NSLAB, SLABC, CH)
    dst3 = edge_index[1].reshape(NW, NCHUNK, CH)
    zrow = jnp.zeros((ZCH, D), jnp.float32)

    DEG = _get_sc_deg()(dst3, zrow).reshape(NW, NPAD).T
    P0 = _get_sc_msg()(inputs, src4, dst4, zrow)
    h1 = _tc_dense(inputs, P0, DEG, W_self0, W_neigh0, b0, relu=True)
    P1 = _get_sc_msg()(h1, src4, dst4, zrow)
    out = _tc_dense(h1, P1, DEG, W_self1, W_neigh1, b1, relu=False)
    return out
